# baseline (speedup 1.0000x reference)
import functools

import jax
import jax.numpy as jnp
from jax import lax
from jax.experimental import pallas as pl
from jax.experimental.pallas import tpu as pltpu
from jax.experimental.pallas import tpu_sc as plsc

BATCH = 16384
D = 64
PW = 128
FOLD = 500096
PROWS = FOLD
NBLK = FOLD // PW
INBLK = 1000000 // PW
NC = 2
NS = 16
NW = NC * NS
BPW = BATCH // NW
HALF = BPW // 2
CHUNK = 128
NCHUNK = HALF // CHUNK


def _fold_body(x1_ref, x2_ref, o_ref):
    o_ref[...] = jnp.concatenate(
        [x1_ref[...].T, x2_ref[...].T], axis=1)


@jax.jit
def _fold(table_t):
    return pl.pallas_call(
        _fold_body,
        grid=(NBLK,),
        in_specs=[
            pl.BlockSpec((D, PW), lambda k: (0, k)),
            pl.BlockSpec((D, PW), lambda k: (0, jnp.minimum(k + NBLK, INBLK))),
        ],
        out_specs=pl.BlockSpec((PW, PW), lambda k: (k, 0)),
        out_shape=jax.ShapeDtypeStruct((PROWS, PW), jnp.float32),
        compiler_params=pltpu.CompilerParams(
            dimension_semantics=("arbitrary",)),
    )(table_t, table_t)


def _mf_body(uq_hbm, iq_hbm, uo_hbm, io_hbm, uf_hbm, if_hbm, out_hbm,
             uidx, iidx, uoff, ioff, urows, irows, outv, sem):
    wid = lax.axis_index("s") * NC + lax.axis_index("c")
    lane = lax.iota(jnp.int32, 16)

    for half in range(2):
        hid = wid * 2 + half
        pltpu.sync_copy(uq_hbm.at[hid], uidx)
        pltpu.sync_copy(iq_hbm.at[hid], iidx)
        pltpu.sync_copy(uo_hbm.at[hid], uoff)
        pltpu.sync_copy(io_hbm.at[hid], ioff)

        for j in range(NCHUNK):
            pltpu.async_copy(
                uf_hbm.at[uidx.at[j]],
                urows.at[pl.ds(j * CHUNK, CHUNK)], sem)
            pltpu.async_copy(
                if_hbm.at[iidx.at[j]],
                irows.at[pl.ds(j * CHUNK, CHUNK)], sem)

        pltpu.make_async_copy(
            uf_hbm.at[pl.ds(0, HALF)], urows, sem).wait()
        pltpu.make_async_copy(
            if_hbm.at[pl.ds(0, HALF)], irows, sem).wait()

        def group(g, carry):
            s = g * 16
            rows = s + lane
            cu = uoff[pl.ds(s, 16)]
            ci = ioff[pl.ds(s, 16)]
            accs = [jnp.zeros((16,), jnp.float32) for _ in range(4)]
            for c in range(D):
                u = plsc.load_gather(urows, [rows, cu])
                v = plsc.load_gather(irows, [rows, ci])
                accs[c % 4] = accs[c % 4] + u * v
                cu = cu + 1
                ci = ci + 1
            outv[pl.ds(s, 16)] = (accs[0] + accs[1]) + (accs[2] + accs[3])
            return carry

        lax.fori_loop(0, HALF // 16, group, 0)

        pltpu.sync_copy(outv, out_hbm.at[pl.ds(hid * HALF, HALF)])


@jax.jit
def _mf(uq3d, iq3d, uo2d, io2d, ufp, ifp):
    mesh = plsc.VectorSubcoreMesh(core_axis_name="c", subcore_axis_name="s")
    kern = functools.partial(
        pl.kernel,
        mesh=mesh,
        compiler_params=pltpu.CompilerParams(
            needs_layout_passes=False, use_tc_tiling_on_sc=True),
        out_type=jax.ShapeDtypeStruct((BATCH,), jnp.float32),
        scratch_types=[
            pltpu.VMEM((NCHUNK, CHUNK), jnp.int32),
            pltpu.VMEM((NCHUNK, CHUNK), jnp.int32),
            pltpu.VMEM((HALF,), jnp.int32),
            pltpu.VMEM((HALF,), jnp.int32),
            pltpu.VMEM((HALF, PW), jnp.float32),
            pltpu.VMEM((HALF, PW), jnp.float32),
            pltpu.VMEM((HALF,), jnp.float32),
            pltpu.SemaphoreType.DMA,
        ],
    )(_mf_body)
    return kern(uq3d, iq3d, uo2d, io2d, ufp, ifp)


def kernel(data, user_factors, item_factors):
    users = data[:, 0].astype(jnp.int32)
    items = data[:, 1].astype(jnp.int32)
    uq3d = (users % FOLD).reshape(NW * 2, NCHUNK, CHUNK)
    iq3d = (items % FOLD).reshape(NW * 2, NCHUNK, CHUNK)
    uo2d = ((users // FOLD) * D).reshape(NW * 2, HALF)
    io2d = ((items // FOLD) * D).reshape(NW * 2, HALF)
    return _mf(uq3d, iq3d, uo2d, io2d,
               _fold(user_factors.T), _fold(item_factors.T))


# speedup vs baseline: 6.0241x; 6.0241x over previous
import functools

import jax
import jax.numpy as jnp
from jax import lax
from jax.experimental import pallas as pl
from jax.experimental.pallas import tpu as pltpu
from jax.experimental.pallas import tpu_sc as plsc

BATCH = 16384
D = 64
PW = 128
B = 2048
FOLD = 501760
PROWS = FOLD
NBLK = FOLD // B
INBLK = -(-1000000 // B) - 1
NC = 2
NS = 16
NW = NC * NS
BPW = BATCH // NW
HALF = BPW // 2
CHUNK = 128
NCHUNK = HALF // CHUNK


def _fold_body(x1_ref, x2_ref, o_ref):
    eye = jnp.float32(
        lax.iota(jnp.int32, D)[:, None] == lax.iota(jnp.int32, D)[None, :])
    dims = (((0,), (0,)), ((), ()))
    y1 = lax.dot_general(x1_ref[...], eye, dims,
                         preferred_element_type=jnp.float32)
    y2 = lax.dot_general(x2_ref[...], eye, dims,
                         preferred_element_type=jnp.float32)
    o_ref[...] = jnp.concatenate([y1, y2], axis=1)


@jax.jit
def _fold(table_t):
    return pl.pallas_call(
        _fold_body,
        grid=(NBLK,),
        in_specs=[
            pl.BlockSpec((D, B), lambda k: (0, k)),
            pl.BlockSpec((D, B), lambda k: (0, jnp.minimum(k + NBLK, INBLK))),
        ],
        out_specs=pl.BlockSpec((B, PW), lambda k: (k, 0)),
        out_shape=jax.ShapeDtypeStruct((PROWS, PW), jnp.float32),
        compiler_params=pltpu.CompilerParams(
            dimension_semantics=("arbitrary",)),
    )(table_t, table_t)


def _mf_body(uq_hbm, iq_hbm, uo_hbm, io_hbm, uf_hbm, if_hbm, out_hbm,
             uidx, iidx, uoff, ioff, urows, irows, outv, sem):
    wid = lax.axis_index("s") * NC + lax.axis_index("c")
    lane = lax.iota(jnp.int32, 16)

    for half in range(2):
        hid = wid * 2 + half
        pltpu.sync_copy(uq_hbm.at[hid], uidx)
        pltpu.sync_copy(iq_hbm.at[hid], iidx)
        pltpu.sync_copy(uo_hbm.at[hid], uoff)
        pltpu.sync_copy(io_hbm.at[hid], ioff)

        for j in range(NCHUNK):
            pltpu.async_copy(
                uf_hbm.at[uidx.at[j]],
                urows.at[pl.ds(j * CHUNK, CHUNK)], sem)
            pltpu.async_copy(
                if_hbm.at[iidx.at[j]],
                irows.at[pl.ds(j * CHUNK, CHUNK)], sem)

        pltpu.make_async_copy(
            uf_hbm.at[pl.ds(0, HALF)], urows, sem).wait()
        pltpu.make_async_copy(
            if_hbm.at[pl.ds(0, HALF)], irows, sem).wait()

        def group(g, carry):
            s = g * 16
            rows = s + lane
            cu = uoff[pl.ds(s, 16)]
            ci = ioff[pl.ds(s, 16)]
            accs = [jnp.zeros((16,), jnp.float32) for _ in range(4)]
            for c in range(D):
                u = plsc.load_gather(urows, [rows, cu])
                v = plsc.load_gather(irows, [rows, ci])
                accs[c % 4] = accs[c % 4] + u * v
                cu = cu + 1
                ci = ci + 1
            outv[pl.ds(s, 16)] = (accs[0] + accs[1]) + (accs[2] + accs[3])
            return carry

        lax.fori_loop(0, HALF // 16, group, 0)

        pltpu.sync_copy(outv, out_hbm.at[pl.ds(hid * HALF, HALF)])


@jax.jit
def _mf(uq3d, iq3d, uo2d, io2d, ufp, ifp):
    mesh = plsc.VectorSubcoreMesh(core_axis_name="c", subcore_axis_name="s")
    kern = functools.partial(
        pl.kernel,
        mesh=mesh,
        compiler_params=pltpu.CompilerParams(
            needs_layout_passes=False, use_tc_tiling_on_sc=True),
        out_type=jax.ShapeDtypeStruct((BATCH,), jnp.float32),
        scratch_types=[
            pltpu.VMEM((NCHUNK, CHUNK), jnp.int32),
            pltpu.VMEM((NCHUNK, CHUNK), jnp.int32),
            pltpu.VMEM((HALF,), jnp.int32),
            pltpu.VMEM((HALF,), jnp.int32),
            pltpu.VMEM((HALF, PW), jnp.float32),
            pltpu.VMEM((HALF, PW), jnp.float32),
            pltpu.VMEM((HALF,), jnp.float32),
            pltpu.SemaphoreType.DMA,
        ],
    )(_mf_body)
    return kern(uq3d, iq3d, uo2d, io2d, ufp, ifp)


def kernel(data, user_factors, item_factors):
    users = data[:, 0].astype(jnp.int32)
    items = data[:, 1].astype(jnp.int32)
    uq3d = (users % FOLD).reshape(NW * 2, NCHUNK, CHUNK)
    iq3d = (items % FOLD).reshape(NW * 2, NCHUNK, CHUNK)
    uo2d = ((users // FOLD) * D).reshape(NW * 2, HALF)
    io2d = ((items // FOLD) * D).reshape(NW * 2, HALF)
    return _mf(uq3d, iq3d, uo2d, io2d,
               _fold(user_factors.T), _fold(item_factors.T))


# speedup vs baseline: 7.4048x; 1.2292x over previous
import functools

import jax
import jax.numpy as jnp
from jax import lax
from jax.experimental import pallas as pl
from jax.experimental.pallas import tpu as pltpu
from jax.experimental.pallas import tpu_sc as plsc

BATCH = 16384
D = 64
PW = 128
B = 4096
FOLD = 503808
PROWS = FOLD
NBLK = FOLD // B
INBLK = -(-1000000 // B) - 1
NC = 2
NS = 16
NW = NC * NS
BPW = BATCH // NW
HALF = BPW // 2
CHUNK = 128
NCHUNK = HALF // CHUNK


def _fold_body(x1_ref, x2_ref, o_ref):
    eye = jnp.float32(
        lax.iota(jnp.int32, D)[:, None] == lax.iota(jnp.int32, D)[None, :])
    dims = (((0,), (0,)), ((), ()))
    y1 = lax.dot_general(x1_ref[...], eye, dims,
                         preferred_element_type=jnp.float32)
    y2 = lax.dot_general(x2_ref[...], eye, dims,
                         preferred_element_type=jnp.float32)
    o_ref[...] = jnp.concatenate([y1, y2], axis=1)


@jax.jit
def _fold(table_t):
    return pl.pallas_call(
        _fold_body,
        grid=(NBLK,),
        in_specs=[
            pl.BlockSpec((D, B), lambda k: (0, k)),
            pl.BlockSpec((D, B), lambda k: (0, jnp.minimum(k + NBLK, INBLK))),
        ],
        out_specs=pl.BlockSpec((B, PW), lambda k: (k, 0)),
        out_shape=jax.ShapeDtypeStruct((PROWS, PW), jnp.float32),
        compiler_params=pltpu.CompilerParams(
            dimension_semantics=("arbitrary",),
            fuse_transposed_lhs_in_matmul=True),
    )(table_t, table_t)


def _mf_body(uq_hbm, iq_hbm, uo_hbm, io_hbm, uf_hbm, if_hbm, out_hbm,
             uidx, iidx, uoff, ioff, urows, irows, outv, sem):
    wid = lax.axis_index("s") * NC + lax.axis_index("c")
    lane = lax.iota(jnp.int32, 16)

    for half in range(2):
        hid = wid * 2 + half
        pltpu.sync_copy(uq_hbm.at[hid], uidx)
        pltpu.sync_copy(iq_hbm.at[hid], iidx)
        pltpu.sync_copy(uo_hbm.at[hid], uoff)
        pltpu.sync_copy(io_hbm.at[hid], ioff)

        for j in range(NCHUNK):
            pltpu.async_copy(
                uf_hbm.at[uidx.at[j]],
                urows.at[pl.ds(j * CHUNK, CHUNK)], sem)
            pltpu.async_copy(
                if_hbm.at[iidx.at[j]],
                irows.at[pl.ds(j * CHUNK, CHUNK)], sem)

        pltpu.make_async_copy(
            uf_hbm.at[pl.ds(0, HALF)], urows, sem).wait()
        pltpu.make_async_copy(
            if_hbm.at[pl.ds(0, HALF)], irows, sem).wait()

        def group(g, carry):
            s = g * 16
            rows = s + lane
            cu = uoff[pl.ds(s, 16)]
            ci = ioff[pl.ds(s, 16)]
            accs = [jnp.zeros((16,), jnp.float32) for _ in range(4)]
            for c in range(D):
                u = plsc.load_gather(urows, [rows, cu])
                v = plsc.load_gather(irows, [rows, ci])
                accs[c % 4] = accs[c % 4] + u * v
                cu = cu + 1
                ci = ci + 1
            outv[pl.ds(s, 16)] = (accs[0] + accs[1]) + (accs[2] + accs[3])
            return carry

        lax.fori_loop(0, HALF // 16, group, 0)

        pltpu.sync_copy(outv, out_hbm.at[pl.ds(hid * HALF, HALF)])


@jax.jit
def _mf(uq3d, iq3d, uo2d, io2d, ufp, ifp):
    mesh = plsc.VectorSubcoreMesh(core_axis_name="c", subcore_axis_name="s")
    kern = functools.partial(
        pl.kernel,
        mesh=mesh,
        compiler_params=pltpu.CompilerParams(
            needs_layout_passes=False, use_tc_tiling_on_sc=True),
        out_type=jax.ShapeDtypeStruct((BATCH,), jnp.float32),
        scratch_types=[
            pltpu.VMEM((NCHUNK, CHUNK), jnp.int32),
            pltpu.VMEM((NCHUNK, CHUNK), jnp.int32),
            pltpu.VMEM((HALF,), jnp.int32),
            pltpu.VMEM((HALF,), jnp.int32),
            pltpu.VMEM((HALF, PW), jnp.float32),
            pltpu.VMEM((HALF, PW), jnp.float32),
            pltpu.VMEM((HALF,), jnp.float32),
            pltpu.SemaphoreType.DMA,
        ],
    )(_mf_body)
    return kern(uq3d, iq3d, uo2d, io2d, ufp, ifp)


def kernel(data, user_factors, item_factors):
    users = data[:, 0].astype(jnp.int32)
    items = data[:, 1].astype(jnp.int32)
    uq3d = (users % FOLD).reshape(NW * 2, NCHUNK, CHUNK)
    iq3d = (items % FOLD).reshape(NW * 2, NCHUNK, CHUNK)
    uo2d = ((users // FOLD) * D).reshape(NW * 2, HALF)
    io2d = ((items // FOLD) * D).reshape(NW * 2, HALF)
    return _mf(uq3d, iq3d, uo2d, io2d,
               _fold(user_factors.T), _fold(item_factors.T))


# speedup vs baseline: 8.9395x; 1.2073x over previous
import functools

import jax
import jax.numpy as jnp
from jax import lax
from jax.experimental import pallas as pl
from jax.experimental.pallas import tpu as pltpu
from jax.experimental.pallas import tpu_sc as plsc

BATCH = 16384
D = 64
PW = 128
B = 4096
FOLD = 503808
PROWS = FOLD
NBLK = FOLD // B
INBLK = -(-1000000 // B) - 1
NC = 2
NS = 16
NW = NC * NS
BPW = BATCH // NW
HALF = BPW // 2
CHUNK = 128
NCHUNK = HALF // CHUNK


def _fold_body(x1_ref, x2_ref, o_ref):
    ci = lax.iota(jnp.int32, D)[:, None]
    ji = lax.iota(jnp.int32, 2 * D)[None, :]
    e1 = jnp.bfloat16(ji == ci)
    e2 = jnp.bfloat16(ji == ci + D)
    dims = (((0,), (0,)), ((), ()))
    y1 = lax.dot_general(x1_ref[...].astype(jnp.bfloat16), e1, dims,
                         preferred_element_type=jnp.float32)
    y2 = lax.dot_general(x2_ref[...].astype(jnp.bfloat16), e2, dims,
                         preferred_element_type=jnp.float32)
    o_ref[...] = y1 + y2


@jax.jit
def _fold(table_t):
    return pl.pallas_call(
        _fold_body,
        grid=(NBLK,),
        in_specs=[
            pl.BlockSpec((D, B), lambda k: (0, k)),
            pl.BlockSpec((D, B), lambda k: (0, jnp.minimum(k + NBLK, INBLK))),
        ],
        out_specs=pl.BlockSpec((B, PW), lambda k: (k, 0)),
        out_shape=jax.ShapeDtypeStruct((PROWS, PW), jnp.float32),
        compiler_params=pltpu.CompilerParams(
            dimension_semantics=("arbitrary",),
            fuse_transposed_lhs_in_matmul=True),
    )(table_t, table_t)


def _mf_body(uq_hbm, iq_hbm, uo_hbm, io_hbm, uf_hbm, if_hbm, out_hbm,
             uidx, iidx, uoff, ioff, urows, irows, outv, sem):
    wid = lax.axis_index("s") * NC + lax.axis_index("c")
    lane = lax.iota(jnp.int32, 16)

    for half in range(2):
        hid = wid * 2 + half
        pltpu.sync_copy(uq_hbm.at[hid], uidx)
        pltpu.sync_copy(iq_hbm.at[hid], iidx)
        pltpu.sync_copy(uo_hbm.at[hid], uoff)
        pltpu.sync_copy(io_hbm.at[hid], ioff)

        for j in range(NCHUNK):
            pltpu.async_copy(
                uf_hbm.at[uidx.at[j]],
                urows.at[pl.ds(j * CHUNK, CHUNK)], sem)
            pltpu.async_copy(
                if_hbm.at[iidx.at[j]],
                irows.at[pl.ds(j * CHUNK, CHUNK)], sem)

        pltpu.make_async_copy(
            uf_hbm.at[pl.ds(0, HALF)], urows, sem).wait()
        pltpu.make_async_copy(
            if_hbm.at[pl.ds(0, HALF)], irows, sem).wait()

        def group(g, carry):
            s = g * 16
            rows = s + lane
            cu = uoff[pl.ds(s, 16)]
            ci = ioff[pl.ds(s, 16)]
            accs = [jnp.zeros((16,), jnp.float32) for _ in range(4)]
            for c in range(D):
                u = plsc.load_gather(urows, [rows, cu])
                v = plsc.load_gather(irows, [rows, ci])
                accs[c % 4] = accs[c % 4] + u * v
                cu = cu + 1
                ci = ci + 1
            outv[pl.ds(s, 16)] = (accs[0] + accs[1]) + (accs[2] + accs[3])
            return carry

        lax.fori_loop(0, HALF // 16, group, 0)

        pltpu.sync_copy(outv, out_hbm.at[pl.ds(hid * HALF, HALF)])


@jax.jit
def _mf(uq3d, iq3d, uo2d, io2d, ufp, ifp):
    mesh = plsc.VectorSubcoreMesh(core_axis_name="c", subcore_axis_name="s")
    kern = functools.partial(
        pl.kernel,
        mesh=mesh,
        compiler_params=pltpu.CompilerParams(
            needs_layout_passes=False, use_tc_tiling_on_sc=True),
        out_type=jax.ShapeDtypeStruct((BATCH,), jnp.float32),
        scratch_types=[
            pltpu.VMEM((NCHUNK, CHUNK), jnp.int32),
            pltpu.VMEM((NCHUNK, CHUNK), jnp.int32),
            pltpu.VMEM((HALF,), jnp.int32),
            pltpu.VMEM((HALF,), jnp.int32),
            pltpu.VMEM((HALF, PW), jnp.float32),
            pltpu.VMEM((HALF, PW), jnp.float32),
            pltpu.VMEM((HALF,), jnp.float32),
            pltpu.SemaphoreType.DMA,
        ],
    )(_mf_body)
    return kern(uq3d, iq3d, uo2d, io2d, ufp, ifp)


def kernel(data, user_factors, item_factors):
    users = data[:, 0].astype(jnp.int32)
    items = data[:, 1].astype(jnp.int32)
    uq3d = (users % FOLD).reshape(NW * 2, NCHUNK, CHUNK)
    iq3d = (items % FOLD).reshape(NW * 2, NCHUNK, CHUNK)
    uo2d = ((users // FOLD) * D).reshape(NW * 2, HALF)
    io2d = ((items // FOLD) * D).reshape(NW * 2, HALF)
    return _mf(uq3d, iq3d, uo2d, io2d,
               _fold(user_factors.T), _fold(item_factors.T))


# speedup vs baseline: 12.6288x; 1.4127x over previous
import functools

import jax
import jax.numpy as jnp
from jax import lax
from jax.experimental import pallas as pl
from jax.experimental.pallas import tpu as pltpu
from jax.experimental.pallas import tpu_sc as plsc

BATCH = 16384
D = 64
PW = 128
B = 4096
FOLD = 258048
PROWS = FOLD
NBLK = FOLD // B
INBLK = -(-1000000 // B) - 1
NC = 2
NS = 16
NW = NC * NS
BPW = BATCH // NW
HALF = BPW // 2
CHUNK = 128
NCHUNK = HALF // CHUNK
INTER = plsc.PackFormat.INTERLEAVED


def _fold_body(x0_ref, x1_ref, x2_ref, x3_ref, o_ref):
    ci = lax.iota(jnp.int32, D)[:, None]
    ji = lax.iota(jnp.int32, 2 * D)[None, :]
    e1 = jnp.bfloat16(ji == ci)
    e2 = jnp.bfloat16(ji == ci + D)
    dims = (((0,), (0,)), ((), ()))

    def t(x_ref, e):
        return lax.dot_general(x_ref[...].astype(jnp.bfloat16), e, dims,
                               preferred_element_type=jnp.float32)

    ylo = t(x0_ref, e1) + t(x2_ref, e2)
    yhi = t(x1_ref, e1) + t(x3_ref, e2)
    ulo = lax.bitcast_convert_type(ylo, jnp.uint32)
    uhi = lax.bitcast_convert_type(yhi, jnp.uint32)
    word = (uhi & jnp.uint32(0xFFFF0000)) | (ulo >> 16)
    o_ref[...] = lax.bitcast_convert_type(word, jnp.float32)


@jax.jit
def _fold(table_t):
    specs = [
        pl.BlockSpec(
            (D, B),
            functools.partial(
                lambda a, k: (0, jnp.minimum(k + a * NBLK, INBLK)), a))
        for a in range(4)
    ]
    return pl.pallas_call(
        _fold_body,
        grid=(NBLK,),
        in_specs=specs,
        out_specs=pl.BlockSpec((B, PW), lambda k: (k, 0)),
        out_shape=jax.ShapeDtypeStruct((PROWS, PW), jnp.float32),
        compiler_params=pltpu.CompilerParams(
            dimension_semantics=("arbitrary",),
            fuse_transposed_lhs_in_matmul=True),
    )(table_t, table_t, table_t, table_t)


def _mf_body(uq_hbm, iq_hbm, uo_hbm, io_hbm, up_hbm, ip_hbm,
             uf_hbm, if_hbm, out_hbm,
             uidx, iidx, uoff, ioff, upar, ipar, urows, irows, outv, sem):
    wid = lax.axis_index("s") * NC + lax.axis_index("c")
    lane = lax.iota(jnp.int32, 16)

    for half in range(2):
        hid = wid * 2 + half
        pltpu.sync_copy(uq_hbm.at[hid], uidx)
        pltpu.sync_copy(iq_hbm.at[hid], iidx)
        pltpu.sync_copy(uo_hbm.at[hid], uoff)
        pltpu.sync_copy(io_hbm.at[hid], ioff)
        pltpu.sync_copy(up_hbm.at[hid], upar)
        pltpu.sync_copy(ip_hbm.at[hid], ipar)

        for j in range(NCHUNK):
            pltpu.async_copy(
                uf_hbm.at[uidx.at[j]],
                urows.at[pl.ds(j * CHUNK, CHUNK)], sem)
            pltpu.async_copy(
                if_hbm.at[iidx.at[j]],
                irows.at[pl.ds(j * CHUNK, CHUNK)], sem)

        pltpu.make_async_copy(
            uf_hbm.at[pl.ds(0, HALF)], urows, sem).wait()
        pltpu.make_async_copy(
            if_hbm.at[pl.ds(0, HALF)], irows, sem).wait()

        def group(g, carry):
            s = g * 16
            rows = s + lane
            cu = uoff[pl.ds(s, 16)]
            ci = ioff[pl.ds(s, 16)]
            mu = upar[pl.ds(s, 16)] == 1
            mi = ipar[pl.ds(s, 16)] == 1
            accs = [jnp.zeros((16,), jnp.float32) for _ in range(4)]
            for c in range(D):
                gu = plsc.load_gather(urows, [rows, cu])
                gv = plsc.load_gather(irows, [rows, ci])
                ulo, uhi = plsc.unpack(
                    plsc.bitcast(gu, jnp.bfloat16), format=INTER)
                vlo, vhi = plsc.unpack(
                    plsc.bitcast(gv, jnp.bfloat16), format=INTER)
                u = jnp.where(mu, uhi, ulo)
                v = jnp.where(mi, vhi, vlo)
                accs[c % 4] = accs[c % 4] + u * v
                cu = cu + 1
                ci = ci + 1
            outv[pl.ds(s, 16)] = (accs[0] + accs[1]) + (accs[2] + accs[3])
            return carry

        lax.fori_loop(0, HALF // 16, group, 0)

        pltpu.sync_copy(outv, out_hbm.at[pl.ds(hid * HALF, HALF)])


@jax.jit
def _mf(uq3d, iq3d, uo2d, io2d, up2d, ip2d, ufp, ifp):
    mesh = plsc.VectorSubcoreMesh(core_axis_name="c", subcore_axis_name="s")
    kern = functools.partial(
        pl.kernel,
        mesh=mesh,
        compiler_params=pltpu.CompilerParams(
            needs_layout_passes=False, use_tc_tiling_on_sc=True),
        out_type=jax.ShapeDtypeStruct((BATCH,), jnp.float32),
        scratch_types=[
            pltpu.VMEM((NCHUNK, CHUNK), jnp.int32),
            pltpu.VMEM((NCHUNK, CHUNK), jnp.int32),
            pltpu.VMEM((HALF,), jnp.int32),
            pltpu.VMEM((HALF,), jnp.int32),
            pltpu.VMEM((HALF,), jnp.int32),
            pltpu.VMEM((HALF,), jnp.int32),
            pltpu.VMEM((HALF, PW), jnp.float32),
            pltpu.VMEM((HALF, PW), jnp.float32),
            pltpu.VMEM((HALF,), jnp.float32),
            pltpu.SemaphoreType.DMA,
        ],
    )(_mf_body)
    return kern(uq3d, iq3d, uo2d, io2d, up2d, ip2d, ufp, ifp)


def kernel(data, user_factors, item_factors):
    users = data[:, 0].astype(jnp.int32)
    items = data[:, 1].astype(jnp.int32)
    ua = users // FOLD
    ia = items // FOLD
    uq3d = (users % FOLD).reshape(NW * 2, NCHUNK, CHUNK)
    iq3d = (items % FOLD).reshape(NW * 2, NCHUNK, CHUNK)
    uo2d = ((ua >> 1) * D).reshape(NW * 2, HALF)
    io2d = ((ia >> 1) * D).reshape(NW * 2, HALF)
    up2d = (ua & 1).reshape(NW * 2, HALF)
    ip2d = (ia & 1).reshape(NW * 2, HALF)
    return _mf(uq3d, iq3d, uo2d, io2d, up2d, ip2d,
               _fold(user_factors.T), _fold(item_factors.T))


# speedup vs baseline: 13.5368x; 1.0719x over previous
import functools

import jax
import jax.numpy as jnp
from jax import lax
from jax.experimental import pallas as pl
from jax.experimental.pallas import tpu as pltpu
from jax.experimental.pallas import tpu_sc as plsc

BATCH = 16384
D = 64
PW = 128
B = 8192
FOLD = 262144
PROWS = FOLD
NBLK = FOLD // B
INBLK = -(-1000000 // B) - 1
NC = 2
NS = 16
NW = NC * NS
BPW = BATCH // NW
HALF = BPW // 2
CHUNK = 128
NCHUNK = HALF // CHUNK
INTER = plsc.PackFormat.INTERLEAVED


def _fold_body(x0_ref, x1_ref, x2_ref, x3_ref, o_ref):
    ci = lax.iota(jnp.int32, D)[:, None]
    ji = lax.iota(jnp.int32, 2 * D)[None, :]
    e1 = jnp.bfloat16(ji == ci)
    e2 = jnp.bfloat16(ji == ci + D)
    dims = (((0,), (0,)), ((), ()))

    def t(x_ref, e):
        return lax.dot_general(x_ref[...].astype(jnp.bfloat16), e, dims,
                               preferred_element_type=jnp.float32)

    ylo = t(x0_ref, e1) + t(x2_ref, e2)
    yhi = t(x1_ref, e1) + t(x3_ref, e2)
    ulo = lax.bitcast_convert_type(ylo, jnp.uint32)
    uhi = lax.bitcast_convert_type(yhi, jnp.uint32)
    word = (uhi & jnp.uint32(0xFFFF0000)) | (ulo >> 16)
    o_ref[...] = lax.bitcast_convert_type(word, jnp.float32)


@jax.jit
def _fold(table_t):
    specs = [
        pl.BlockSpec(
            (D, B),
            functools.partial(
                lambda a, k: (0, jnp.minimum(k + a * NBLK, INBLK)), a))
        for a in range(4)
    ]
    return pl.pallas_call(
        _fold_body,
        grid=(NBLK,),
        in_specs=specs,
        out_specs=pl.BlockSpec((B, PW), lambda k: (k, 0)),
        out_shape=jax.ShapeDtypeStruct((PROWS, PW), jnp.float32),
        compiler_params=pltpu.CompilerParams(
            dimension_semantics=("arbitrary",),
            fuse_transposed_lhs_in_matmul=True),
    )(table_t, table_t, table_t, table_t)


def _mf_body(uq_hbm, iq_hbm, uo_hbm, io_hbm, up_hbm, ip_hbm,
             uf_hbm, if_hbm, out_hbm,
             uidx, iidx, uoff, ioff, upar, ipar, urows, irows, outv, sem):
    wid = lax.axis_index("s") * NC + lax.axis_index("c")
    lane = lax.iota(jnp.int32, 16)

    for half in range(2):
        hid = wid * 2 + half
        pltpu.sync_copy(uq_hbm.at[hid], uidx)
        pltpu.sync_copy(iq_hbm.at[hid], iidx)
        pltpu.sync_copy(uo_hbm.at[hid], uoff)
        pltpu.sync_copy(io_hbm.at[hid], ioff)
        pltpu.sync_copy(up_hbm.at[hid], upar)
        pltpu.sync_copy(ip_hbm.at[hid], ipar)

        for j in range(NCHUNK):
            pltpu.async_copy(
                uf_hbm.at[uidx.at[j]],
                urows.at[pl.ds(j * CHUNK, CHUNK)], sem)
            pltpu.async_copy(
                if_hbm.at[iidx.at[j]],
                irows.at[pl.ds(j * CHUNK, CHUNK)], sem)

        pltpu.make_async_copy(
            uf_hbm.at[pl.ds(0, HALF)], urows, sem).wait()
        pltpu.make_async_copy(
            if_hbm.at[pl.ds(0, HALF)], irows, sem).wait()

        def group(g, carry):
            s = g * 16
            rows = s + lane
            cu = uoff[pl.ds(s, 16)]
            ci = ioff[pl.ds(s, 16)]
            mu = upar[pl.ds(s, 16)] == 1
            mi = ipar[pl.ds(s, 16)] == 1
            accs = [jnp.zeros((16,), jnp.float32) for _ in range(4)]
            for c in range(D):
                gu = plsc.load_gather(urows, [rows, cu])
                gv = plsc.load_gather(irows, [rows, ci])
                ulo, uhi = plsc.unpack(
                    plsc.bitcast(gu, jnp.bfloat16), format=INTER)
                vlo, vhi = plsc.unpack(
                    plsc.bitcast(gv, jnp.bfloat16), format=INTER)
                u = jnp.where(mu, uhi, ulo)
                v = jnp.where(mi, vhi, vlo)
                accs[c % 4] = accs[c % 4] + u * v
                cu = cu + 1
                ci = ci + 1
            outv[pl.ds(s, 16)] = (accs[0] + accs[1]) + (accs[2] + accs[3])
            return carry

        lax.fori_loop(0, HALF // 16, group, 0)

        pltpu.sync_copy(outv, out_hbm.at[pl.ds(hid * HALF, HALF)])


@jax.jit
def _mf(uq3d, iq3d, uo2d, io2d, up2d, ip2d, ufp, ifp):
    mesh = plsc.VectorSubcoreMesh(core_axis_name="c", subcore_axis_name="s")
    kern = functools.partial(
        pl.kernel,
        mesh=mesh,
        compiler_params=pltpu.CompilerParams(
            needs_layout_passes=False, use_tc_tiling_on_sc=True),
        out_type=jax.ShapeDtypeStruct((BATCH,), jnp.float32),
        scratch_types=[
            pltpu.VMEM((NCHUNK, CHUNK), jnp.int32),
            pltpu.VMEM((NCHUNK, CHUNK), jnp.int32),
            pltpu.VMEM((HALF,), jnp.int32),
            pltpu.VMEM((HALF,), jnp.int32),
            pltpu.VMEM((HALF,), jnp.int32),
            pltpu.VMEM((HALF,), jnp.int32),
            pltpu.VMEM((HALF, PW), jnp.float32),
            pltpu.VMEM((HALF, PW), jnp.float32),
            pltpu.VMEM((HALF,), jnp.float32),
            pltpu.SemaphoreType.DMA,
        ],
    )(_mf_body)
    return kern(uq3d, iq3d, uo2d, io2d, up2d, ip2d, ufp, ifp)


def kernel(data, user_factors, item_factors):
    users = data[:, 0].astype(jnp.int32)
    items = data[:, 1].astype(jnp.int32)
    ua = users // FOLD
    ia = items // FOLD
    uq3d = (users % FOLD).reshape(NW * 2, NCHUNK, CHUNK)
    iq3d = (items % FOLD).reshape(NW * 2, NCHUNK, CHUNK)
    uo2d = ((ua >> 1) * D).reshape(NW * 2, HALF)
    io2d = ((ia >> 1) * D).reshape(NW * 2, HALF)
    up2d = (ua & 1).reshape(NW * 2, HALF)
    ip2d = (ia & 1).reshape(NW * 2, HALF)
    return _mf(uq3d, iq3d, uo2d, io2d, up2d, ip2d,
               _fold(user_factors.T), _fold(item_factors.T))


# speedup vs baseline: 13.8758x; 1.0250x over previous
import functools

import jax
import jax.numpy as jnp
from jax import lax
from jax.experimental import pallas as pl
from jax.experimental.pallas import tpu as pltpu
from jax.experimental.pallas import tpu_sc as plsc

BATCH = 16384
D = 64
PW = 128
B = 8192
FOLD = 262144
PROWS = FOLD
NBLK = FOLD // B
INBLK = -(-1000000 // B) - 1
NC = 2
NS = 16
NW = NC * NS
BPW = BATCH // NW
HALF = BPW // 2
CHUNK = 128
NCHUNK = HALF // CHUNK
INTER = plsc.PackFormat.INTERLEAVED


def _fold_body(x0_ref, x1_ref, x2_ref, x3_ref, o_ref):
    ci = lax.iota(jnp.int32, D)[:, None]
    ji = lax.iota(jnp.int32, 2 * D)[None, :]
    e1 = jnp.bfloat16(ji == ci)
    e2 = jnp.bfloat16(ji == ci + D)
    dims = (((0,), (0,)), ((), ()))

    def t(x_ref, e):
        return lax.dot_general(x_ref[...].astype(jnp.bfloat16), e, dims,
                               preferred_element_type=jnp.float32)

    ylo = t(x0_ref, e1) + t(x2_ref, e2)
    yhi = t(x1_ref, e1) + t(x3_ref, e2)
    ulo = lax.bitcast_convert_type(ylo, jnp.uint32)
    uhi = lax.bitcast_convert_type(yhi, jnp.uint32)
    word = (uhi & jnp.uint32(0xFFFF0000)) | (ulo >> 16)
    o_ref[...] = lax.bitcast_convert_type(word, jnp.float32)


@jax.jit
def _fold(table_t):
    specs = [
        pl.BlockSpec(
            (D, B),
            functools.partial(
                lambda a, k: (0, jnp.minimum(k + a * NBLK, INBLK)), a))
        for a in range(4)
    ]
    return pl.pallas_call(
        _fold_body,
        grid=(NBLK,),
        in_specs=specs,
        out_specs=pl.BlockSpec((B, PW), lambda k: (k, 0)),
        out_shape=jax.ShapeDtypeStruct((PROWS, PW), jnp.float32),
        compiler_params=pltpu.CompilerParams(
            dimension_semantics=("arbitrary",),
            fuse_transposed_lhs_in_matmul=True),
    )(table_t, table_t, table_t, table_t)


QB = 128


def _mf_body(meta_hbm, uf_hbm, if_hbm, out_hbm,
             meta, urow2, irow2, outv, sem0, sem1):
    wid = lax.axis_index("s") * NC + lax.axis_index("c")
    lane = lax.iota(jnp.int32, 16)
    sems = [sem0, sem1]

    pltpu.sync_copy(meta_hbm.at[wid], meta)

    def fire(q):
        slot = q % 2
        pltpu.async_copy(
            uf_hbm.at[meta.at[q]], urow2.at[slot], sems[slot])
        pltpu.async_copy(
            if_hbm.at[meta.at[4 + q]], irow2.at[slot], sems[slot])

    fire(0)
    for q in range(4):
        if q < 3:
            fire(q + 1)
        slot = q % 2
        pltpu.make_async_copy(
            uf_hbm.at[pl.ds(0, QB)], urow2.at[slot], sems[slot]).wait()
        pltpu.make_async_copy(
            if_hbm.at[pl.ds(0, QB)], irow2.at[slot], sems[slot]).wait()
        urows = urow2.at[slot]
        irows = irow2.at[slot]

        def group(g, carry):
            s = g * 16
            rows = s + lane
            cu = meta[8 + q, pl.ds(s, 16)]
            ci = meta[12 + q, pl.ds(s, 16)]
            mu = meta[16 + q, pl.ds(s, 16)] == 1
            mi = meta[20 + q, pl.ds(s, 16)] == 1
            accs = [jnp.zeros((16,), jnp.float32) for _ in range(4)]
            for c in range(D):
                gu = plsc.load_gather(urows, [rows, cu])
                gv = plsc.load_gather(irows, [rows, ci])
                ulo, uhi = plsc.unpack(
                    plsc.bitcast(gu, jnp.bfloat16), format=INTER)
                vlo, vhi = plsc.unpack(
                    plsc.bitcast(gv, jnp.bfloat16), format=INTER)
                u = jnp.where(mu, uhi, ulo)
                v = jnp.where(mi, vhi, vlo)
                accs[c % 4] = accs[c % 4] + u * v
                cu = cu + 1
                ci = ci + 1
            outv[pl.ds(q * QB + s, 16)] = (
                (accs[0] + accs[1]) + (accs[2] + accs[3]))
            return carry

        lax.fori_loop(0, QB // 16, group, 0)

    pltpu.sync_copy(outv, out_hbm.at[pl.ds(wid * BPW, BPW)])


@jax.jit
def _mf(meta, ufp, ifp):
    mesh = plsc.VectorSubcoreMesh(core_axis_name="c", subcore_axis_name="s")
    kern = functools.partial(
        pl.kernel,
        mesh=mesh,
        compiler_params=pltpu.CompilerParams(
            needs_layout_passes=False, use_tc_tiling_on_sc=True),
        out_type=jax.ShapeDtypeStruct((BATCH,), jnp.float32),
        scratch_types=[
            pltpu.VMEM((24, QB), jnp.int32),
            pltpu.VMEM((2, QB, PW), jnp.float32),
            pltpu.VMEM((2, QB, PW), jnp.float32),
            pltpu.VMEM((BPW,), jnp.float32),
            pltpu.SemaphoreType.DMA,
            pltpu.SemaphoreType.DMA,
        ],
    )(_mf_body)
    return kern(meta, ufp, ifp)


def kernel(data, user_factors, item_factors):
    users = data[:, 0].astype(jnp.int32)
    items = data[:, 1].astype(jnp.int32)
    ua = users // FOLD
    ia = items // FOLD
    meta = jnp.concatenate([
        (users % FOLD).reshape(NW, 4, QB),
        (items % FOLD).reshape(NW, 4, QB),
        ((ua >> 1) * D).reshape(NW, 4, QB),
        ((ia >> 1) * D).reshape(NW, 4, QB),
        (ua & 1).reshape(NW, 4, QB),
        (ia & 1).reshape(NW, 4, QB),
    ], axis=1)
    return _mf(meta, _fold(user_factors.T), _fold(item_factors.T))


# speedup vs baseline: 13.9080x; 1.0023x over previous
import functools

import jax
import jax.numpy as jnp
from jax import lax
from jax.experimental import pallas as pl
from jax.experimental.pallas import tpu as pltpu
from jax.experimental.pallas import tpu_sc as plsc

BATCH = 16384
D = 64
PW = 128
B = 12288
FOLD = 258048
PROWS = FOLD
NBLK = FOLD // B
INBLK = -(-1000000 // B) - 1
NC = 2
NS = 16
NW = NC * NS
BPW = BATCH // NW
HALF = BPW // 2
CHUNK = 128
NCHUNK = HALF // CHUNK
INTER = plsc.PackFormat.INTERLEAVED


def _fold_body(x0_ref, x1_ref, x2_ref, x3_ref, o_ref):
    ci = lax.iota(jnp.int32, D)[:, None]
    ji = lax.iota(jnp.int32, 2 * D)[None, :]
    e1 = jnp.bfloat16(ji == ci)
    e2 = jnp.bfloat16(ji == ci + D)
    dims = (((0,), (0,)), ((), ()))

    def t(x_ref, e):
        return lax.dot_general(x_ref[...].astype(jnp.bfloat16), e, dims,
                               preferred_element_type=jnp.float32)

    ylo = t(x0_ref, e1) + t(x2_ref, e2)
    yhi = t(x1_ref, e1) + t(x3_ref, e2)
    ulo = lax.bitcast_convert_type(ylo, jnp.uint32)
    uhi = lax.bitcast_convert_type(yhi, jnp.uint32)
    word = (uhi & jnp.uint32(0xFFFF0000)) | (ulo >> 16)
    o_ref[...] = lax.bitcast_convert_type(word, jnp.float32)


@jax.jit
def _fold(table_t):
    specs = [
        pl.BlockSpec(
            (D, B),
            functools.partial(
                lambda a, k: (0, jnp.minimum(k + a * NBLK, INBLK)), a))
        for a in range(4)
    ]
    return pl.pallas_call(
        _fold_body,
        grid=(NBLK,),
        in_specs=specs,
        out_specs=pl.BlockSpec((B, PW), lambda k: (k, 0)),
        out_shape=jax.ShapeDtypeStruct((PROWS, PW), jnp.float32),
        compiler_params=pltpu.CompilerParams(
            dimension_semantics=("arbitrary",),
            fuse_transposed_lhs_in_matmul=True),
    )(table_t, table_t, table_t, table_t)


QB = 128


def _mf_body(meta_hbm, uf_hbm, if_hbm, out_hbm,
             meta, urow2, irow2, outv, sem0, sem1):
    wid = lax.axis_index("s") * NC + lax.axis_index("c")
    lane = lax.iota(jnp.int32, 16)
    sems = [sem0, sem1]

    pltpu.sync_copy(meta_hbm.at[wid], meta)

    def fire(q):
        slot = q % 2
        pltpu.async_copy(
            uf_hbm.at[meta.at[q]], urow2.at[slot], sems[slot])
        pltpu.async_copy(
            if_hbm.at[meta.at[4 + q]], irow2.at[slot], sems[slot])

    fire(0)
    for q in range(4):
        if q < 3:
            fire(q + 1)
        slot = q % 2
        pltpu.make_async_copy(
            uf_hbm.at[pl.ds(0, QB)], urow2.at[slot], sems[slot]).wait()
        pltpu.make_async_copy(
            if_hbm.at[pl.ds(0, QB)], irow2.at[slot], sems[slot]).wait()
        urows = urow2.at[slot]
        irows = irow2.at[slot]

        def group(g, carry):
            s = g * 16
            rows = s + lane
            cu = meta[8 + q, pl.ds(s, 16)]
            ci = meta[12 + q, pl.ds(s, 16)]
            mu = meta[16 + q, pl.ds(s, 16)] == 1
            mi = meta[20 + q, pl.ds(s, 16)] == 1
            accs = [jnp.zeros((16,), jnp.float32) for _ in range(4)]
            for c in range(D):
                gu = plsc.load_gather(urows, [rows, cu])
                gv = plsc.load_gather(irows, [rows, ci])
                ulo, uhi = plsc.unpack(
                    plsc.bitcast(gu, jnp.bfloat16), format=INTER)
                vlo, vhi = plsc.unpack(
                    plsc.bitcast(gv, jnp.bfloat16), format=INTER)
                u = jnp.where(mu, uhi, ulo)
                v = jnp.where(mi, vhi, vlo)
                accs[c % 4] = accs[c % 4] + u * v
                cu = cu + 1
                ci = ci + 1
            outv[pl.ds(q * QB + s, 16)] = (
                (accs[0] + accs[1]) + (accs[2] + accs[3]))
            return carry

        lax.fori_loop(0, QB // 16, group, 0)

    pltpu.sync_copy(outv, out_hbm.at[pl.ds(wid * BPW, BPW)])


@jax.jit
def _mf(meta, ufp, ifp):
    mesh = plsc.VectorSubcoreMesh(core_axis_name="c", subcore_axis_name="s")
    kern = functools.partial(
        pl.kernel,
        mesh=mesh,
        compiler_params=pltpu.CompilerParams(
            needs_layout_passes=False, use_tc_tiling_on_sc=True),
        out_type=jax.ShapeDtypeStruct((BATCH,), jnp.float32),
        scratch_types=[
            pltpu.VMEM((24, QB), jnp.int32),
            pltpu.VMEM((2, QB, PW), jnp.float32),
            pltpu.VMEM((2, QB, PW), jnp.float32),
            pltpu.VMEM((BPW,), jnp.float32),
            pltpu.SemaphoreType.DMA,
            pltpu.SemaphoreType.DMA,
        ],
    )(_mf_body)
    return kern(meta, ufp, ifp)


def kernel(data, user_factors, item_factors):
    users = data[:, 0].astype(jnp.int32)
    items = data[:, 1].astype(jnp.int32)
    ua = users // FOLD
    ia = items // FOLD
    meta = jnp.concatenate([
        (users % FOLD).reshape(NW, 4, QB),
        (items % FOLD).reshape(NW, 4, QB),
        ((ua >> 1) * D).reshape(NW, 4, QB),
        ((ia >> 1) * D).reshape(NW, 4, QB),
        (ua & 1).reshape(NW, 4, QB),
        (ia & 1).reshape(NW, 4, QB),
    ], axis=1)
    return _mf(meta, _fold(user_factors.T), _fold(item_factors.T))
